# Initial kernel scaffold; baseline (speedup 1.0000x reference)
import jax
import jax.numpy as jnp
from jax.experimental import pallas as pl


def kernel(y_pred, y_true):
    raise NotImplementedError("write your pallas kernel here")


# speedup vs baseline: 26.9669x; 26.9669x over previous
import functools

import jax
import jax.numpy as jnp
from jax.experimental import pallas as pl

_ALPHA = 0.007
_BETA = 0.016
_OMEGA_O = 0.57
_OMEGA_T = 0.41
_NO_VALUE = -999.0
_EDGE_W = (1.0, 0.98, 0.97, 0.96, 0.95)

_B, _T, _C, _H, _W = 4, 12, 1, 512, 512
_NUMEL = _B * _C * _H * _W
_K90_LO = 943717
_K13 = 349525

_IMIN = -2147483648


def _f32_key(x):
    u = jax.lax.bitcast_convert_type(x, jnp.int32)
    return jnp.where(u >= 0, u, _IMIN - u)


def _key_to_f32(k):
    u = jnp.where(k >= 0, k, _IMIN - k)
    return jax.lax.bitcast_convert_type(u.astype(jnp.int32), jnp.float32)


def _select_rank(key, k90, k13):
    def body(i, carry):
        a90, a13 = carry
        b = 31 - i
        low = jax.lax.shift_left(jnp.int32(1), b) - 1
        bit = jax.lax.shift_left(jnp.int32(1), b)
        t90 = jax.lax.bitwise_xor(jax.lax.bitwise_or(a90, low), jnp.int32(_IMIN))
        t13 = jax.lax.bitwise_xor(jax.lax.bitwise_or(a13, low), jnp.int32(_IMIN))
        c90 = jnp.sum((key <= t90).astype(jnp.int32))
        c13 = jnp.sum((key <= t13).astype(jnp.int32))
        a90 = jnp.where(c90 < k90 + 1, jax.lax.bitwise_or(a90, bit), a90)
        a13 = jnp.where(c13 < k13 + 1, jax.lax.bitwise_or(a13, bit), a13)
        return a90, a13

    z = jnp.int32(0)
    a90_u, a13_u = jax.lax.fori_loop(0, 32, body, (z, z))
    key90 = jax.lax.bitwise_xor(a90_u, jnp.int32(_IMIN))
    key13 = jax.lax.bitwise_xor(a13_u, jnp.int32(_IMIN))
    return key90, key13


def _edge_weight(idx):
    e = jnp.zeros_like(idx, dtype=jnp.float32)
    for i, w in enumerate(_EDGE_W):
        e = e + jnp.where(idx == i, w, 0.0) + jnp.where(idx == (_H - 1 - i), w, 0.0)
    return e


def _min_weight(m):
    e = jnp.zeros_like(m, dtype=jnp.float32)
    for i, w in enumerate(_EDGE_W):
        e = e + jnp.where(m == i, w, 0.0)
    return e


def _loss_kernel(yp_ref, yt_ref, out_ref):
    yt = yt_ref[...]
    yp = yp_ref[...]

    key = _f32_key(yt)
    key90a, key13 = _select_rank(key, _K90_LO, _K13)
    c_a = jnp.sum((key <= key90a).astype(jnp.int32))
    nxt = jnp.min(jnp.where(key > key90a, key, jnp.int32(2147483647)))
    key90b = jnp.where(c_a >= _K90_LO + 2, key90a, nxt)
    va = _key_to_f32(key90a)
    vb = _key_to_f32(key90b)
    q90 = va + (vb - va) * jnp.float32(0.5)
    q13 = _key_to_f32(key13)

    h = jax.lax.broadcasted_iota(jnp.int32, yt.shape, 3)
    w = jax.lax.broadcasted_iota(jnp.int32, yt.shape, 4)
    diag = (h == w) | (h + w == _H - 1)
    bmask = _edge_weight(h) + _edge_weight(w) + jnp.where(
        diag, _min_weight(jnp.minimum(h, _H - 1 - h)), 0.0)

    diff = jnp.abs(yt - yp)
    no_value = yt == _NO_VALUE
    outlier = yt > q90
    normal = jnp.logical_not(no_value | outlier)
    over = (yp >= yt) & normal
    under = (yp < yt) & normal
    torr = (yt >= q13) & normal
    wts = _ALPHA * jnp.exp(_BETA * yt)
    sq = (yt - yp) * (yt - yp)
    wsq = wts * sq

    zero = jnp.float32(0.0)
    s_low = jnp.sum(jnp.where(no_value, diff, zero))
    s_out = jnp.sum(jnp.where(outlier, diff, zero))
    s_bnd = jnp.sum(bmask * diff)
    s_over = jnp.sum(jnp.where(over, diff, zero))
    s_under = jnp.sum(jnp.where(under, diff, zero))
    s_tover = jnp.sum(jnp.where(torr & over, wsq, zero))
    s_tunder = jnp.sum(jnp.where(torr & under, wsq, zero))
    s_abs = jnp.sum(jnp.where(no_value, zero, diff))
    s_cnt = jnp.sum(jnp.where(no_value, zero, jnp.float32(1.0)))

    inv_n = jnp.float32(1.0 / _NUMEL)
    loss_t = (
        _OMEGA_O * s_low
        + (1.0 - _OMEGA_O) * s_out
        + (1.0 - _OMEGA_O) * s_bnd
        + (1.0 - _OMEGA_O) * s_over + _OMEGA_O * s_under
        + (1.0 - _OMEGA_T) * s_tover + _OMEGA_T * s_tunder
    ) * inv_n

    r = jax.lax.broadcasted_iota(jnp.int32, (1, 8, 128), 1)
    c = jax.lax.broadcasted_iota(jnp.int32, (1, 8, 128), 2)
    first = r == 0
    tile = (jnp.where(first & (c == 0), loss_t, zero)
            + jnp.where(first & (c == 1), s_abs, zero)
            + jnp.where(first & (c == 2), s_cnt, zero))
    out_ref[...] = tile


@jax.jit
def kernel(y_pred, y_true):
    block = (_B, 1, _C, _H, _W)
    partials = pl.pallas_call(
        _loss_kernel,
        grid=(_T,),
        in_specs=[
            pl.BlockSpec(block, lambda t: (0, t, 0, 0, 0)),
            pl.BlockSpec(block, lambda t: (0, t, 0, 0, 0)),
        ],
        out_specs=pl.BlockSpec((1, 8, 128), lambda t: (t, 0, 0)),
        out_shape=jax.ShapeDtypeStruct((_T, 8, 128), jnp.float32),
    )(y_pred, y_true)

    losses = partials[:, 0, 0]
    s_abs = partials[:, 0, 1]
    s_cnt = partials[:, 0, 2]
    seasons = ((0, 1, 11), (2, 3, 4), (5, 6, 7), (8, 9, 10))
    seasonal = jnp.float32(0.0)
    for idx in seasons:
        ii = jnp.asarray(idx)
        seasonal = seasonal + jnp.sum(s_abs[ii]) / jnp.sum(s_cnt[ii])
    return jnp.mean(losses) + seasonal
